# baseline (speedup 1.0000x reference)
import jax
import jax.numpy as jnp
from jax.experimental import pallas as pl
from jax.experimental.pallas import tpu as pltpu

D = 768
E = 64
TILE = 4096


def _gate_body(x_ref, sim_ref, gates_ref, mask_ref,
               out_ref, topk_ref, sn_ref, thr_ref):
    i = pl.program_id(0)

    @pl.when(i == 0)
    def _():
        sim = sim_ref[:]
        cn = jnp.sqrt(jnp.sum(sim * sim, axis=0, keepdims=True))
        sn_ref[:] = (sim / jnp.maximum(cn, 1e-12)).astype(jnp.bfloat16)
        thr_ref[:] = jnp.transpose(
            jnp.where(mask_ref[:] != 0.0, gates_ref[:], jnp.inf))

    xb = x_ref[:]
    rn2 = jnp.sum(xb * xb, axis=1, keepdims=True)
    inv = jax.lax.rsqrt(jnp.maximum(rn2, 1e-24))
    xn = (xb * inv).astype(jnp.bfloat16)
    d = jnp.dot(xn, sn_ref[:], preferred_element_type=jnp.float32)
    dt = jnp.transpose(d)
    act = jnp.where(dt > thr_ref[:], 1.0, 0.0)
    out_ref[:] = act
    ones = jnp.ones((1, E), dtype=jnp.float32)
    tk = jnp.dot(ones, act, preferred_element_type=jnp.float32)
    topk_ref[:] = tk.astype(jnp.int32)


def kernel(x, sim_matrix, gates, experts_mask, temperature):
    del temperature
    n_tok = x.shape[0]
    grid = (n_tok // TILE,)
    out_t, topk = pl.pallas_call(
        _gate_body,
        grid=grid,
        in_specs=[
            pl.BlockSpec((TILE, D), lambda i: (i, 0)),
            pl.BlockSpec((D, E), lambda i: (0, 0)),
            pl.BlockSpec((1, E), lambda i: (0, 0)),
            pl.BlockSpec((1, E), lambda i: (0, 0)),
        ],
        out_specs=[
            pl.BlockSpec((E, TILE), lambda i: (0, i)),
            pl.BlockSpec((1, TILE), lambda i: (0, i)),
        ],
        out_shape=[
            jax.ShapeDtypeStruct((E, n_tok), jnp.float32),
            jax.ShapeDtypeStruct((1, n_tok), jnp.int32),
        ],
        scratch_shapes=[pltpu.VMEM((D, E), jnp.bfloat16),
                        pltpu.VMEM((E, 1), jnp.float32)],
    )(x, sim_matrix, gates.reshape(1, E), experts_mask.reshape(1, E))
    return out_t.T, topk.reshape(n_tok)


# speedup vs baseline: 1.0558x; 1.0558x over previous
import jax
import jax.numpy as jnp
from jax.experimental import pallas as pl
from jax.experimental.pallas import tpu as pltpu

D = 768
E = 64
TILE = 4096


def _gate_body(x_ref, sim_ref, gates_ref, mask_ref,
               out_ref, topk_ref, sn_ref, thr_ref):
    i = pl.program_id(0)

    @pl.when(i == 0)
    def _():
        simt = sim_ref[:]
        cn = jnp.sqrt(jnp.sum(simt * simt, axis=1, keepdims=True))
        snt = simt / jnp.maximum(cn, 1e-12)
        sn_ref[:] = jnp.transpose(snt).astype(jnp.bfloat16)
        thr_ref[:] = jnp.transpose(
            jnp.where(mask_ref[:] != 0.0, gates_ref[:], jnp.inf))

    xb = x_ref[:]
    rn2 = jnp.sum(xb * xb, axis=1, keepdims=True)
    inv = jax.lax.rsqrt(jnp.maximum(rn2, 1e-24))
    xn = (xb * inv).astype(jnp.bfloat16)
    d = jnp.dot(xn, sn_ref[:], preferred_element_type=jnp.float32)
    dt = jnp.transpose(d)
    act = jnp.where(dt > thr_ref[:], 1.0, 0.0)
    out_ref[:] = act
    ones = jnp.ones((1, E), dtype=jnp.float32)
    tk = jnp.dot(ones, act, preferred_element_type=jnp.float32)
    topk_ref[:] = tk.astype(jnp.int32)


def kernel(x, sim_matrix, gates, experts_mask, temperature):
    del temperature
    n_tok = x.shape[0]
    grid = (n_tok // TILE,)
    out_t, topk = pl.pallas_call(
        _gate_body,
        grid=grid,
        in_specs=[
            pl.BlockSpec((TILE, D), lambda i: (i, 0)),
            pl.BlockSpec((E, D), lambda i: (0, 0)),
            pl.BlockSpec((1, E), lambda i: (0, 0)),
            pl.BlockSpec((1, E), lambda i: (0, 0)),
        ],
        out_specs=[
            pl.BlockSpec((E, TILE), lambda i: (0, i)),
            pl.BlockSpec((1, TILE), lambda i: (0, i)),
        ],
        out_shape=[
            jax.ShapeDtypeStruct((E, n_tok), jnp.float32),
            jax.ShapeDtypeStruct((1, n_tok), jnp.int32),
        ],
        scratch_shapes=[pltpu.VMEM((D, E), jnp.bfloat16),
                        pltpu.VMEM((E, 1), jnp.float32)],
    )(x, sim_matrix.T, gates.reshape(1, E), experts_mask.reshape(1, E))
    return out_t.T, topk.reshape(n_tok)
